# Initial kernel scaffold; baseline (speedup 1.0000x reference)
import jax
import jax.numpy as jnp
from jax.experimental import pallas as pl


def kernel(x, edge_index, W1a, b1a, W1b, b1b, W2a, b2a, W2b, b2b):
    raise NotImplementedError("write your pallas kernel here")


# speedup vs baseline: 3.6150x; 3.6150x over previous
import functools

import jax
import jax.numpy as jnp
from jax import lax
from jax.experimental import pallas as pl
from jax.experimental.pallas import tpu as pltpu
from jax.experimental.pallas import tpu_sc as plsc

_N = 10000
_E = 320000
_C = 128
_H = 256

_NC = 2
_NS = 16
_B = 80
_NP = 10240
_RPS = _NP // _NS


def _sc_segment_sum(table, src, dst, zeros, *, feat_split):
  edges_per_core = _E if feat_split else _E // _NC
  eps = edges_per_core // _NS
  nb = eps // _B
  assert eps % _B == 0

  mesh = plsc.VectorSubcoreMesh(core_axis_name="c", subcore_axis_name="s")

  @functools.partial(
      pl.kernel,
      out_type=jax.ShapeDtypeStruct((_NC, _NP, _C), jnp.float32),
      mesh=mesh,
      scratch_types=[
          pltpu.VMEM((_B,), jnp.int32),
          pltpu.VMEM((_B,), jnp.int32),
          pltpu.VMEM((_B,), jnp.int32),
          pltpu.VMEM((_B, _C), jnp.float32),
          pltpu.VMEM_SHARED((_NP, _C), jnp.float32),
          pltpu.SemaphoreType.DMA,
      ],
  )
  def k(table_h, src_h, dst_h, zero_h, out_h,
        src_v, src2_v, dst_v, rows_v, acc, sem):
    c = lax.axis_index("c")
    s = lax.axis_index("s")

    pltpu.sync_copy(zero_h.at[pl.ds(s * _RPS, _RPS)],
                    acc.at[pl.ds(s * _RPS, _RPS)])
    plsc.subcore_barrier()

    base = s * eps if feat_split else c * edges_per_core + s * eps

    def body(i, carry):
      off = pl.multiple_of(base + i * _B, 8)
      pltpu.sync_copy(src_h.at[pl.ds(off, _B)], src_v)
      if feat_split:
        for j in range(_B // 16):
          v = src_v[pl.ds(j * 16, 16)]
          src2_v[pl.ds(j * 16, 16)] = v * 2 + c
        idx_ref = src2_v
      else:
        idx_ref = src_v
      pltpu.async_copy(table_h.at[idx_ref], rows_v, sem).wait()
      pltpu.sync_copy(dst_h.at[pl.ds(off, _B)], dst_v)
      pltpu.sync_copy(rows_v, acc.at[dst_v], add=True)
      return carry

    lax.fori_loop(0, nb, body, 0)
    plsc.subcore_barrier()
    pltpu.sync_copy(acc.at[pl.ds(s * _RPS, _RPS)],
                    out_h.at[c, pl.ds(s * _RPS, _RPS)])

  return k(table, src, dst, zeros)


_BLK = 400


def _mlp1_body(x_ref, p_ref, wa_ref, ba_ref, wb_ref, bb_ref, h_ref):
  t = x_ref[...] + p_ref[0] + p_ref[1]
  a = jnp.maximum(
      jnp.dot(t, wa_ref[...], preferred_element_type=jnp.float32)
      + ba_ref[...], 0.0)
  h = jnp.maximum(
      jnp.dot(a, wb_ref[...], preferred_element_type=jnp.float32)
      + bb_ref[...], 0.0)
  h_ref[...] = h


def _mlp1(x, p, W1a, b1a, W1b, b1b):
  return pl.pallas_call(
      _mlp1_body,
      grid=(_N // _BLK,),
      in_specs=[
          pl.BlockSpec((_BLK, _C), lambda i: (i, 0)),
          pl.BlockSpec((_NC, _BLK, _C), lambda i: (0, i, 0)),
          pl.BlockSpec((_C, _H), lambda i: (0, 0)),
          pl.BlockSpec((1, _H), lambda i: (0, 0)),
          pl.BlockSpec((_H, _H), lambda i: (0, 0)),
          pl.BlockSpec((1, _H), lambda i: (0, 0)),
      ],
      out_specs=pl.BlockSpec((_BLK, _H), lambda i: (i, 0)),
      out_shape=jax.ShapeDtypeStruct((_N, _H), jnp.float32),
  )(x, p, W1a, b1a.reshape(1, _H), W1b, b1b.reshape(1, _H))


def _mlp2_body(h_ref, p_ref, wa_ref, ba_ref, wb_ref, bb_ref, o_ref):
  t = h_ref[...] + jnp.concatenate([p_ref[0], p_ref[1]], axis=1)
  z = jnp.maximum(
      jnp.dot(t, wa_ref[...], preferred_element_type=jnp.float32)
      + ba_ref[...], 0.0)
  u = jnp.dot(z, wb_ref[...], preferred_element_type=jnp.float32) + bb_ref[...]
  o_ref[...] = 1.0 / (1.0 + jnp.exp(-u))


def _mlp2(h, p, W2a, b2a, W2b, b2b):
  return pl.pallas_call(
      _mlp2_body,
      grid=(_N // _BLK,),
      in_specs=[
          pl.BlockSpec((_BLK, _H), lambda i: (i, 0)),
          pl.BlockSpec((_NC, _BLK, _C), lambda i: (0, i, 0)),
          pl.BlockSpec((_H, _H), lambda i: (0, 0)),
          pl.BlockSpec((1, _H), lambda i: (0, 0)),
          pl.BlockSpec((_H, _C), lambda i: (0, 0)),
          pl.BlockSpec((1, _C), lambda i: (0, 0)),
      ],
      out_specs=pl.BlockSpec((_BLK, _C), lambda i: (i, 0)),
      out_shape=jax.ShapeDtypeStruct((_N, _C), jnp.float32),
  )(h, p, W2a, b2a.reshape(1, _H), W2b, b2b.reshape(1, _C))


def kernel(x, edge_index, W1a, b1a, W1b, b1b, W2a, b2a, W2b, b2b):
  src = edge_index[0].astype(jnp.int32)
  dst = edge_index[1].astype(jnp.int32)
  zeros = jnp.zeros((_NP, _C), jnp.float32)

  p1 = _sc_segment_sum(x, src, dst, zeros, feat_split=False)[:, :_N]
  h = _mlp1(x, p1, W1a, b1a, W1b, b1b)
  p2 = _sc_segment_sum(h.reshape(2 * _N, _C), src, dst, zeros,
                       feat_split=True)[:, :_N]
  return _mlp2(h, p2, W2a, b2a, W2b, b2b)
